# Initial kernel scaffold; baseline (speedup 1.0000x reference)
import jax
import jax.numpy as jnp
from jax.experimental import pallas as pl


def kernel(features, coords):
    raise NotImplementedError("write your pallas kernel here")


# speedup vs baseline: 4.1975x; 4.1975x over previous
import jax
import jax.numpy as jnp
from jax import lax
from jax.experimental import pallas as pl
from jax.experimental.pallas import tpu as pltpu

B, N, D = 4, 576, 384
LAMBDA_SPATIAL = 0.5
SIGMA_SQ = 100.0 * 100.0


def _body(feat_ref, featT_ref, coord_ref, coordT_ref, out_r_ref, out_o_ref,
          sim_ref):
    iota_l = lax.broadcasted_iota(jnp.int32, (1, N), 1)

    for b in range(B):
        f = feat_ref[b]
        fT = featT_ref[b]
        s = jnp.sum(f * f, axis=1, keepdims=True)
        norm = jnp.maximum(jnp.sqrt(s), 1e-12)
        fn = f / norm
        sT = jnp.sum(fT * fT, axis=0, keepdims=True)
        normT = jnp.maximum(jnp.sqrt(sT), 1e-12)
        fnT = fT / normT
        sem = jnp.dot(fn, fnT, preferred_element_type=jnp.float32)
        x_col = coord_ref[b][:, 0:1]
        y_col = coord_ref[b][:, 1:2]
        x_row = coordT_ref[b][0:1, :]
        y_row = coordT_ref[b][1:2, :]
        dx = x_col - x_row
        dy = y_col - y_row
        dist = jnp.sqrt(dx * dx + dy * dy)
        spat = jnp.exp(-(dist * dist) / SIGMA_SQ)
        sim_ref[b] = sem + LAMBDA_SPATIAL * spat

    init = []
    for b in range(B):
        conn = jnp.sum(sim_ref[b], axis=1, keepdims=True)
        mconn = jnp.max(conn)
        iota_s = lax.broadcasted_iota(jnp.int32, (N, 1), 0)
        start = jnp.min(jnp.where(conn == mconn, iota_s, N))
        vis = jnp.where(iota_l == start, jnp.int32(1), jnp.int32(0))
        orow = jnp.where(iota_l == 0, start, jnp.int32(0))
        out_r_ref[b, pl.ds(0, 1), :] = feat_ref[b, pl.ds(start, 1), :]
        init += [vis, orow, start]

    def step(i, carry):
        new = []
        for b in range(B):
            vis, orow, cur = carry[3 * b], carry[3 * b + 1], carry[3 * b + 2]
            row = sim_ref[b, pl.ds(cur, 1), :]
            masked = jnp.where(vis > 0, -jnp.inf, row)
            m = jnp.max(masked)
            nxt = jnp.min(jnp.where(masked == m, iota_l, jnp.int32(N)))
            vis = jnp.where(iota_l == nxt, jnp.int32(1), vis)
            orow = jnp.where(iota_l == i, nxt, orow)
            out_r_ref[b, pl.ds(i, 1), :] = feat_ref[b, pl.ds(nxt, 1), :]
            new += [vis, orow, nxt]
        return tuple(new)

    carry = lax.fori_loop(1, N, step, tuple(init), unroll=False)
    for b in range(B):
        out_o_ref[pl.ds(b, 1), :] = carry[3 * b + 1]


def kernel(features, coords):
    featT = jnp.swapaxes(features, 1, 2)
    coordT = jnp.swapaxes(coords, 1, 2)
    reordered, orders = pl.pallas_call(
        _body,
        out_shape=(
            jax.ShapeDtypeStruct((B, N, D), jnp.float32),
            jax.ShapeDtypeStruct((B, N), jnp.int32),
        ),
        scratch_shapes=[pltpu.VMEM((B, N, N), jnp.float32)],
    )(features, featT, coords, coordT)
    return reordered, orders
